# baseline (speedup 1.0000x reference)
import functools

import jax
import jax.numpy as jnp
from jax import lax
from jax.experimental import pallas as pl
from jax.experimental.pallas import tpu as pltpu
from jax.experimental.pallas import tpu_sc as plsc

H = 128
NRBF = 20
CUT = 5.0
EPS = 1e-8
NG = 64
N_E = 320000
N_N = 10000
E_PAD = 327680
N_PAD = 10240

CH = 128
NROWS = E_PAD // CH
GW_ROWS = NROWS // 32
SC_ROWS = NROWS // 16
NPT = N_PAD // 16
BE = 512
BN = 1024


def _sc_gather2():
    mesh = plsc.VectorSubcoreMesh(core_axis_name="c", subcore_axis_name="s")

    @functools.partial(
        pl.kernel, mesh=mesh,
        out_type=[jax.ShapeDtypeStruct((E_PAD, H), jnp.int32)] * 2,
        scratch_types=[
            pltpu.VMEM((GW_ROWS, CH), jnp.int32),
        ] + [pltpu.VMEM((CH, H), jnp.int32)] * 4
          + [pltpu.SemaphoreType.DMA] * 8,
    )
    def k(t0, t1, idx_hbm, o0, o1, idx_v, b0, b1, b2, b3, *sems):
        bufs = (b0, b1, b2, b3)
        gsems = sems[:4]
        wsems = sems[4:]
        wid = lax.axis_index("s") * 2 + lax.axis_index("c")
        row0 = wid * GW_ROWS
        pltpu.sync_copy(idx_hbm.at[pl.ds(row0, GW_ROWS)], idx_v)

        for tb, ob in ((t0, o0), (t1, o1)):
            def body(i0, _, tb=tb, ob=ob):
                for b in range(4):
                    @pl.when(i0 > 0)
                    def _(b=b, ob=ob):
                        pltpu.make_async_copy(
                            bufs[b], ob.at[pl.ds((row0 + i0 + b) * CH, CH)],
                            wsems[b]).wait()
                hs = [pltpu.async_copy(tb.at[idx_v.at[i0 + b]],
                                       bufs[b], gsems[b]) for b in range(4)]
                for b in range(4):
                    hs[b].wait()
                    pltpu.async_copy(
                        bufs[b], ob.at[pl.ds((row0 + i0 + b) * CH, CH)],
                        wsems[b])
                return ()
            lax.fori_loop(0, GW_ROWS // 4, lambda i, c: body(i * 4, c), ())
            for b in range(4):
                pltpu.make_async_copy(
                    bufs[b], ob.at[pl.ds(row0 * CH, CH)], wsems[b]).wait()

    return k


SC_ROWS_T = NROWS // 32


def _sc_scatter4():
    mesh = plsc.VectorSubcoreMesh(core_axis_name="c", subcore_axis_name="s")

    @functools.partial(
        pl.kernel, mesh=mesh,
        out_type=[jax.ShapeDtypeStruct((2, N_PAD, H), jnp.float32)] * 4,
        scratch_types=[
            pltpu.VMEM((SC_ROWS_T, CH), jnp.int32),
            pltpu.VMEM((CH, H), jnp.float32),
            pltpu.VMEM((CH, H), jnp.float32),
            pltpu.VMEM((32, H), jnp.float32),
            pltpu.VMEM_SHARED((N_PAD, H), jnp.float32),
            pltpu.SemaphoreType.DMA,
            pltpu.SemaphoreType.DMA,
        ],
    )
    def k(v0, v1, v2, v3, idx_hbm, o0, o1, o2, o3,
          idx_v, rb0, rb1, zbuf, acc, sm0, sm1):
        bufs = (rb0, rb1)
        sems = (sm0, sm1)
        c = lax.axis_index("c")
        s = lax.axis_index("s")
        row0 = c * (NROWS // 2) + s * SC_ROWS_T
        pltpu.sync_copy(idx_hbm.at[pl.ds(row0, SC_ROWS_T)], idx_v)

        def zfill(t, _):
            i = t // 8
            j = (t % 8) * 16
            zbuf[i, pl.ds(j, 16)] = jnp.zeros((16,), jnp.float32)
            return ()
        lax.fori_loop(0, 32 * 8, zfill, ())

        nbase = s * NPT

        def do_array(vals, out):
            def zb(t, _):
                pltpu.sync_copy(zbuf, acc.at[pl.ds(nbase + t * 32, 32)])
                return ()
            lax.fori_loop(0, NPT // 32, zb, ())
            plsc.subcore_barrier()

            def body(t0, _, vals=vals):
                hs = [pltpu.async_copy(
                          vals.at[pl.ds((row0 + t0 + b) * CH, CH)],
                          bufs[b], sems[b])
                      for b in range(2)]
                for b in range(2):
                    hs[b].wait()
                    pltpu.sync_copy(bufs[b], acc.at[idx_v.at[t0 + b]],
                                    add=True)
                return ()
            lax.fori_loop(0, SC_ROWS_T // 2, lambda i, cc: body(i * 2, cc), ())
            plsc.subcore_barrier()
            pltpu.sync_copy(acc.at[pl.ds(nbase, NPT)],
                            out.at[c].at[pl.ds(nbase, NPT)])

        do_array(v0, o0)
        do_array(v1, o1)
        do_array(v2, o2)
        do_array(v3, o3)

    return k


def _silu(x):
    return x * (1.0 / (1.0 + jnp.exp(-x)))


def _edge_body(has_v, refs):
    if has_v:
        (ef, rt, p1, p2, w1t, b1, w2t, b2, wst, bs, wvt, bv,
         ms_o, a0_o, a1_o, a2_o) = refs
    else:
        (ef, rt, embp, w1t, b1, w2t, b2, wst, bs, wvt, bv,
         ms_o, a0_o, a1_o, a2_o) = refs
    e = ef[...]
    u0 = e[:, 0:1]
    u1 = e[:, 1:2]
    u2 = e[:, 2:3]
    if not has_v:
        kcol = lax.broadcasted_iota(jnp.int32, (BE, H), 1).astype(jnp.float32)
        onehot = (e[:, 4:5] == kcol).astype(jnp.float32)
        sj_v = jnp.dot(onehot, embp[...], preferred_element_type=jnp.float32)
    rr = rt[...]
    x = rr * (jnp.pi / CUT)
    c1 = jnp.cos(x)
    s1 = jnp.sin(x)
    cv = jnp.where(rr < CUT, 0.5 * (c1 + 1.0), 0.0)
    scale = cv / rr
    two_c1 = 2.0 * c1
    p_prev = jnp.zeros((1, BE), jnp.float32)
    p_cur = s1 * scale
    rows = [p_cur]
    for _ in range(NRBF - 1):
        p_prev, p_cur = p_cur, two_c1 * p_cur - p_prev
        rows.append(p_cur)
    rows.append(jnp.zeros((12, BE), jnp.float32))
    rbf_t = jnp.concatenate(rows, axis=0)
    h1 = _silu(lax.dot_general(rbf_t, w1t[...],
                               (((0,), (0,)), ((), ())),
                               preferred_element_type=jnp.float32)
               + b1[...])
    w = jnp.dot(h1, w2t[...], preferred_element_type=jnp.float32) + b2[...]
    if has_v:
        def unpack(p):
            ui = lax.bitcast_convert_type(p[...], jnp.uint32)
            hi = lax.bitcast_convert_type((ui >> 16).astype(jnp.uint16),
                                          jnp.bfloat16).astype(jnp.float32)
            lo = lax.bitcast_convert_type((ui & 0xFFFF).astype(jnp.uint16),
                                          jnp.bfloat16).astype(jnp.float32)
            return hi, lo
        sj_v, vj0 = unpack(p1)
        vj1, vj2 = unpack(p2)
    ms = (jnp.dot(sj_v, wst[...], preferred_element_type=jnp.float32)
          + bs[...]) * w
    if has_v:
        proj = u0 * vj0 + u1 * vj1 + u2 * vj2
        mv = (jnp.dot(proj, wvt[...], preferred_element_type=jnp.float32)
              + bv[...]) * w
    else:
        mv = bv[...] * w
    ms_o[...] = ms
    a0_o[...] = u0 * mv
    a1_o[...] = u1 * mv
    a2_o[...] = u2 * mv


def _tc_edge(has_v, interpret=False):
    nb = E_PAD // BE
    big = pl.BlockSpec((BE, H), lambda t: (t, 0))
    wspec = pl.BlockSpec((H, H), lambda t: (0, 0))
    bspec = pl.BlockSpec((1, H), lambda t: (0, 0))
    rtspec = pl.BlockSpec((1, BE), lambda t: (0, t))
    if has_v:
        ispec = pl.BlockSpec((BE, H), lambda t: (t, 0))
        in_specs = [pl.BlockSpec((BE, 8), lambda t: (t, 0)), rtspec,
                    ispec, ispec]
    else:
        in_specs = [pl.BlockSpec((BE, 8), lambda t: (t, 0)), rtspec, wspec]
    in_specs += [pl.BlockSpec((32, H), lambda t: (0, 0)),
                 bspec, wspec, bspec, wspec, bspec, wspec, bspec]
    return pl.pallas_call(
        lambda *refs: _edge_body(has_v, refs),
        grid=(nb,),
        in_specs=in_specs,
        out_specs=[big, big, big, big],
        out_shape=[jax.ShapeDtypeStruct((E_PAD, H), jnp.float32)] * 4,
        interpret=interpret,
    )


def _node_body(first, *refs):
    (sv, ms, a0, a1, a2,
     us1t, usb1, us2t, usb2, uv1t, uvb1, uv2t, uvb2,
     out, outb1, outb2) = refs

    def mlp(x, w1t, bb1, w2t, bb2):
        h = _silu(jnp.dot(x, w1t[...], preferred_element_type=jnp.float32)
                  + bb1[...])
        return jnp.dot(h, w2t[...], preferred_element_type=jnp.float32) + bb2[...]

    svv = sv[...]
    s = svv[:, 0:H]
    s_n = s + mlp(ms[0] + ms[1], us1t, usb1, us2t, usb2)
    outs = [s_n]
    for d, a in enumerate((a0, a1, a2)):
        upd = mlp(a[0] + a[1], uv1t, uvb1, uv2t, uvb2)
        if first:
            outs.append(upd)
        else:
            outs.append(svv[:, (d + 1) * H:(d + 2) * H] + upd)
    fused = jnp.concatenate(outs, axis=1)
    out[...] = fused
    fb = fused.astype(jnp.bfloat16)
    p16 = [lax.bitcast_convert_type(fb[:, d * H:(d + 1) * H],
                                    jnp.uint16).astype(jnp.uint32)
           for d in range(4)]
    outb1[...] = lax.bitcast_convert_type((p16[0] << 16) | p16[1], jnp.int32)
    outb2[...] = lax.bitcast_convert_type((p16[2] << 16) | p16[3], jnp.int32)


def _tc_node(first, interpret=False):
    nb = N_PAD // BN
    big = pl.BlockSpec((BN, H), lambda t: (t, 0))
    fat = pl.BlockSpec((BN, 4 * H), lambda t: (t, 0))
    wspec = pl.BlockSpec((H, H), lambda t: (0, 0))
    bspec = pl.BlockSpec((1, H), lambda t: (0, 0))
    svspec = big if first else fat
    part = pl.BlockSpec((2, BN, H), lambda t: (0, t, 0))
    return pl.pallas_call(
        functools.partial(_node_body, first),
        grid=(nb,),
        in_specs=[svspec] + [part] * 4 + [wspec, bspec, wspec, bspec] * 2,
        out_specs=[fat, big, big],
        out_shape=[jax.ShapeDtypeStruct((N_PAD, 4 * H), jnp.float32),
                   jax.ShapeDtypeStruct((N_PAD, H), jnp.int32),
                   jax.ShapeDtypeStruct((N_PAD, H), jnp.int32)],
        interpret=interpret,
    )


def _readout_body(s, batch, wrot, brow, out):
    @pl.when(pl.program_id(0) == 0)
    def _():
        out[...] = jnp.zeros_like(out)
    per_atom = (jnp.dot(s[...], wrot[...], preferred_element_type=jnp.float32)
                + brow[...])
    bvec = batch[0]
    gid = lax.broadcasted_iota(jnp.int32, (NG, BN), 0)
    onehot = (gid == bvec).astype(jnp.float32)
    out[...] += jnp.dot(onehot, per_atom, preferred_element_type=jnp.float32)


def _tc_readout(interpret=False):
    nb = N_PAD // BN
    return pl.pallas_call(
        _readout_body,
        grid=(nb,),
        in_specs=[
            pl.BlockSpec((BN, H), lambda t: (t, 0)),
            pl.BlockSpec((1, 1, BN), lambda t: (t, 0, 0)),
            pl.BlockSpec((H, H), lambda t: (0, 0)),
            pl.BlockSpec((1, H), lambda t: (0, 0)),
        ],
        out_specs=pl.BlockSpec((NG, H), lambda t: (0, 0)),
        out_shape=jax.ShapeDtypeStruct((NG, H), jnp.float32),
        interpret=interpret,
    )


def _row(b):
    return b.reshape(1, H)


def kernel(z, pos, edge_index, batch, emb, layers, W_ro, b_ro):
    ei = edge_index[0].astype(jnp.int32)
    ej = edge_index[1].astype(jnp.int32)

    rij = pos[ej] - pos[ei]
    dist = jnp.sqrt(jnp.sum(rij * rij, axis=-1))
    dist_safe = jnp.maximum(dist, EPS)
    unit = rij / dist_safe[:, None]
    zj = z[ej].astype(jnp.float32)
    ef = jnp.concatenate(
        [unit, dist_safe[:, None], zj[:, None],
         jnp.zeros((N_E, 3), jnp.float32)], axis=1)
    ef = jnp.concatenate(
        [ef, jnp.tile(jnp.array([[0, 0, 0, 1, 0, 0, 0, 0]], jnp.float32),
                      (E_PAD - N_E, 1))], axis=0)
    ei2d = jnp.concatenate(
        [ei, jnp.full((E_PAD - N_E,), N_N, jnp.int32)]).reshape(NROWS, CH)
    ej2d = jnp.concatenate(
        [ej, jnp.zeros((E_PAD - N_E,), jnp.int32)]).reshape(NROWS, CH)

    s0 = jnp.zeros((N_PAD, H), jnp.float32).at[:N_N].set(emb[z])

    gather = _sc_gather2()
    scatter4 = _sc_scatter4()
    edge1 = _tc_edge(False)
    edge2 = _tc_edge(True)

    r_t = jnp.concatenate(
        [dist_safe, jnp.ones((E_PAD - N_E,), jnp.float32)]).reshape(1, E_PAD)

    sv = None
    for li, p in enumerate(layers):
        w1t = jnp.zeros((32, H), jnp.float32).at[:NRBF, :].set(p['phi'][0].T)
        wargs = (w1t, _row(p['phi'][1]), p['phi'][2].T, _row(p['phi'][3]),
                 p['Ws_W'].T, _row(p['Ws_b']), p['Wv_W'].T, _row(p['Wv_b']))
        if li == 0:
            emb_pad = jnp.zeros((H, H), jnp.float32).at[:emb.shape[0]].set(emb)
            ms_e, a0_e, a1_e, a2_e = edge1(ef, r_t, emb_pad, *wargs)
        else:
            svj1, svj2 = gather(svb1, svb2, ej2d)
            ms_e, a0_e, a1_e, a2_e = edge2(ef, r_t, svj1, svj2, *wargs)
        MS, A0, A1, A2 = scatter4(ms_e, a0_e, a1_e, a2_e, ei2d)
        svin = s0 if li == 0 else sv
        sv, svb1, svb2 = _tc_node(li == 0)(
            svin, MS, A0, A1, A2,
            p['Us'][0].T, _row(p['Us'][1]), p['Us'][2].T, _row(p['Us'][3]),
            p['Uv'][0].T, _row(p['Uv'][1]), p['Uv'][2].T, _row(p['Uv'][3]))

    wrot = jnp.zeros((H, H), jnp.float32).at[:, :3].set(W_ro.T)
    brow = jnp.zeros((1, H), jnp.float32).at[0, :3].set(b_ro)
    batch3 = jnp.concatenate(
        [batch.astype(jnp.int32), jnp.full((N_PAD - N_N,), NG, jnp.int32)]
    ).reshape(N_PAD // BN, 1, BN)
    pred_pad = _tc_readout()(sv, batch3, wrot, brow)
    return pred_pad[:, :3]


# speedup vs baseline: 1.3949x; 1.3949x over previous
import functools

import jax
import jax.numpy as jnp
from jax import lax
from jax.experimental import pallas as pl
from jax.experimental.pallas import tpu as pltpu
from jax.experimental.pallas import tpu_sc as plsc

H = 128
NRBF = 20
CUT = 5.0
EPS = 1e-8
NG = 64
N_E = 320000
N_N = 10000
E_PAD = 327680
N_PAD = 10240

CH = 128
NROWS = E_PAD // CH
GW_ROWS = NROWS // 32
SC_ROWS = NROWS // 16
NPT = N_PAD // 16
BE = 512
BN = 1024


CH_G = 64
GROWS = E_PAD // CH_G
GW_FAT = GROWS // 32


def _sc_gather_fat():
    mesh = plsc.VectorSubcoreMesh(core_axis_name="c", subcore_axis_name="s")

    @functools.partial(
        pl.kernel, mesh=mesh,
        out_type=jax.ShapeDtypeStruct((E_PAD, 2 * H), jnp.int32),
        scratch_types=[
            pltpu.VMEM((GW_FAT, CH_G), jnp.int32),
        ] + [pltpu.VMEM((CH_G, 2 * H), jnp.int32)] * 4
          + [pltpu.SemaphoreType.DMA] * 8,
    )
    def k(table, idx_hbm, out, idx_v, b0, b1, b2, b3, *sems):
        bufs = (b0, b1, b2, b3)
        gsems = sems[:4]
        wsems = sems[4:]
        wid = lax.axis_index("s") * 2 + lax.axis_index("c")
        row0 = wid * GW_FAT
        pltpu.sync_copy(idx_hbm.at[pl.ds(row0, GW_FAT)], idx_v)

        def body(i0, _):
            for b in range(4):
                @pl.when(i0 > 0)
                def _(b=b):
                    pltpu.make_async_copy(
                        bufs[b], out.at[pl.ds((row0 + i0 + b) * CH_G, CH_G)],
                        wsems[b]).wait()
            hs = [pltpu.async_copy(table.at[idx_v.at[i0 + b]],
                                   bufs[b], gsems[b]) for b in range(4)]
            for b in range(4):
                hs[b].wait()
                pltpu.async_copy(
                    bufs[b], out.at[pl.ds((row0 + i0 + b) * CH_G, CH_G)],
                    wsems[b])
            return ()
        lax.fori_loop(0, GW_FAT // 4, lambda i, c: body(i * 4, c), ())
        for b in range(4):
            pltpu.make_async_copy(
                bufs[b], out.at[pl.ds(row0 * CH_G, CH_G)], wsems[b]).wait()

    return k


GEO_W = 128


def _sc_gather_geo():
    mesh = plsc.VectorSubcoreMesh(core_axis_name="c", subcore_axis_name="s")

    @functools.partial(
        pl.kernel, mesh=mesh,
        out_type=[jax.ShapeDtypeStruct((E_PAD, GEO_W), jnp.float32)] * 2,
        scratch_types=[
            pltpu.VMEM((GW_ROWS, CH), jnp.int32),
        ] + [pltpu.VMEM((CH, GEO_W), jnp.float32)] * 4
          + [pltpu.SemaphoreType.DMA] * 8,
    )
    def k(geo, ei_hbm, ej_hbm, oi, oj, idx_v, b0, b1, b2, b3, *sems):
        bufs = (b0, b1, b2, b3)
        gsems = sems[:4]
        wsems = sems[4:]
        wid = lax.axis_index("s") * 2 + lax.axis_index("c")
        row0 = wid * GW_ROWS
        for idxh, ob in ((ei_hbm, oi), (ej_hbm, oj)):
            pltpu.sync_copy(idxh.at[pl.ds(row0, GW_ROWS)], idx_v)

            def body(i0, _, ob=ob):
                for b in range(4):
                    @pl.when(i0 > 0)
                    def _(b=b, ob=ob):
                        pltpu.make_async_copy(
                            bufs[b], ob.at[pl.ds((row0 + i0 + b) * CH, CH)],
                            wsems[b]).wait()
                hs = [pltpu.async_copy(geo.at[idx_v.at[i0 + b]],
                                       bufs[b], gsems[b]) for b in range(4)]
                for b in range(4):
                    hs[b].wait()
                    pltpu.async_copy(
                        bufs[b], ob.at[pl.ds((row0 + i0 + b) * CH, CH)],
                        wsems[b])
                return ()
            lax.fori_loop(0, GW_ROWS // 4, lambda i, c: body(i * 4, c), ())
            for b in range(4):
                pltpu.make_async_copy(
                    bufs[b], ob.at[pl.ds(row0 * CH, CH)], wsems[b]).wait()

    return k


SC_ROWS_T = NROWS // 32


def _sc_scatter4():
    mesh = plsc.VectorSubcoreMesh(core_axis_name="c", subcore_axis_name="s")

    @functools.partial(
        pl.kernel, mesh=mesh,
        out_type=[jax.ShapeDtypeStruct((2, N_PAD, H), jnp.float32)] * 4,
        scratch_types=[
            pltpu.VMEM((SC_ROWS_T, CH), jnp.int32),
            pltpu.VMEM((CH, H), jnp.float32),
            pltpu.VMEM((CH, H), jnp.float32),
            pltpu.VMEM((32, H), jnp.float32),
            pltpu.VMEM_SHARED((N_PAD, H), jnp.float32),
            pltpu.SemaphoreType.DMA,
            pltpu.SemaphoreType.DMA,
        ],
    )
    def k(v0, v1, v2, v3, idx_hbm, o0, o1, o2, o3,
          idx_v, rb0, rb1, zbuf, acc, sm0, sm1):
        bufs = (rb0, rb1)
        sems = (sm0, sm1)
        c = lax.axis_index("c")
        s = lax.axis_index("s")
        row0 = c * (NROWS // 2) + s * SC_ROWS_T
        pltpu.sync_copy(idx_hbm.at[pl.ds(row0, SC_ROWS_T)], idx_v)

        def zfill(t, _):
            i = t // 8
            j = (t % 8) * 16
            zbuf[i, pl.ds(j, 16)] = jnp.zeros((16,), jnp.float32)
            return ()
        lax.fori_loop(0, 32 * 8, zfill, ())

        nbase = s * NPT

        def do_array(vals, out):
            def zb(t, _):
                pltpu.sync_copy(zbuf, acc.at[pl.ds(nbase + t * 32, 32)])
                return ()
            lax.fori_loop(0, NPT // 32, zb, ())
            plsc.subcore_barrier()

            def body(t0, _, vals=vals):
                hs = [pltpu.async_copy(
                          vals.at[pl.ds((row0 + t0 + b) * CH, CH)],
                          bufs[b], sems[b])
                      for b in range(2)]
                for b in range(2):
                    hs[b].wait()
                    pltpu.sync_copy(bufs[b], acc.at[idx_v.at[t0 + b]],
                                    add=True)
                return ()
            lax.fori_loop(0, SC_ROWS_T // 2, lambda i, cc: body(i * 2, cc), ())
            plsc.subcore_barrier()
            pltpu.sync_copy(acc.at[pl.ds(nbase, NPT)],
                            out.at[c].at[pl.ds(nbase, NPT)])

        do_array(v0, o0)
        do_array(v1, o1)
        do_array(v2, o2)
        do_array(v3, o3)

    return k


def _silu(x):
    return x * (1.0 / (1.0 + jnp.exp(-x)))


def _geom_body(gi, gj, ef_o, rt_o):
    a = gi[...]
    b = gj[...]
    dx = b[:, 0:1] - a[:, 0:1]
    dy = b[:, 1:2] - a[:, 1:2]
    dz = b[:, 2:3] - a[:, 2:3]
    dist = jnp.sqrt(dx * dx + dy * dy + dz * dz)
    dsafe = jnp.maximum(dist, EPS)
    inv = 1.0 / dsafe
    ef_o[...] = jnp.concatenate(
        [dx * inv, dy * inv, dz * inv, dsafe, b[:, 3:4],
         jnp.zeros((BE, 3), jnp.float32)], axis=1)
    rt_o[...] = jnp.transpose(dsafe)


def _tc_geom(interpret=False):
    nb = E_PAD // BE
    gspec = pl.BlockSpec((BE, GEO_W), lambda t: (t, 0))
    return pl.pallas_call(
        _geom_body,
        grid=(nb,),
        in_specs=[gspec, gspec],
        out_specs=[pl.BlockSpec((BE, 8), lambda t: (t, 0)),
                   pl.BlockSpec((1, BE), lambda t: (0, t))],
        out_shape=[jax.ShapeDtypeStruct((E_PAD, 8), jnp.float32),
                   jax.ShapeDtypeStruct((1, E_PAD), jnp.float32)],
        interpret=interpret,
    )


def _edge_body(has_v, refs):
    if has_v:
        (ef, rt, pk, w1t, b1, w2t, b2, wst, bs, wvt, bv,
         ms_o, a0_o, a1_o, a2_o) = refs
    else:
        (ef, rt, embp, w1t, b1, w2t, b2, wst, bs, wvt, bv,
         ms_o, a0_o, a1_o, a2_o) = refs
    e = ef[...]
    u0 = e[:, 0:1]
    u1 = e[:, 1:2]
    u2 = e[:, 2:3]
    if not has_v:
        kcol = lax.broadcasted_iota(jnp.int32, (BE, H), 1).astype(jnp.float32)
        onehot = (e[:, 4:5] == kcol).astype(jnp.float32)
        sj_v = jnp.dot(onehot, embp[...], preferred_element_type=jnp.float32)
    rr = rt[...]
    x = rr * (jnp.pi / CUT)
    c1 = jnp.cos(x)
    s1 = jnp.sin(x)
    cv = jnp.where(rr < CUT, 0.5 * (c1 + 1.0), 0.0)
    scale = cv / rr
    two_c1 = 2.0 * c1
    p_prev = jnp.zeros((1, BE), jnp.float32)
    p_cur = s1 * scale
    rows = [p_cur]
    for _ in range(NRBF - 1):
        p_prev, p_cur = p_cur, two_c1 * p_cur - p_prev
        rows.append(p_cur)
    rows.append(jnp.zeros((12, BE), jnp.float32))
    rbf_t = jnp.concatenate(rows, axis=0)
    h1 = _silu(lax.dot_general(rbf_t, w1t[...],
                               (((0,), (0,)), ((), ())),
                               preferred_element_type=jnp.float32)
               + b1[...])
    w = jnp.dot(h1, w2t[...], preferred_element_type=jnp.float32) + b2[...]
    if has_v:
        ui = lax.bitcast_convert_type(pk[...], jnp.uint32)
        hi = lax.bitcast_convert_type((ui >> 16).astype(jnp.uint16),
                                      jnp.bfloat16).astype(jnp.float32)
        lo = lax.bitcast_convert_type((ui & 0xFFFF).astype(jnp.uint16),
                                      jnp.bfloat16).astype(jnp.float32)
        sj_v = hi[:, :H]
    ms = (jnp.dot(sj_v, wst[...], preferred_element_type=jnp.float32)
          + bs[...]) * w
    if has_v:
        proj = u0 * hi[:, H:] + u1 * lo[:, :H] + u2 * lo[:, H:]
        mv = (jnp.dot(proj, wvt[...], preferred_element_type=jnp.float32)
              + bv[...]) * w
    else:
        mv = bv[...] * w
    ms_o[...] = ms
    a0_o[...] = u0 * mv
    a1_o[...] = u1 * mv
    a2_o[...] = u2 * mv


def _tc_edge(has_v, interpret=False):
    nb = E_PAD // BE
    big = pl.BlockSpec((BE, H), lambda t: (t, 0))
    wspec = pl.BlockSpec((H, H), lambda t: (0, 0))
    bspec = pl.BlockSpec((1, H), lambda t: (0, 0))
    rtspec = pl.BlockSpec((1, BE), lambda t: (0, t))
    if has_v:
        in_specs = [pl.BlockSpec((BE, 8), lambda t: (t, 0)), rtspec,
                    pl.BlockSpec((BE, 2 * H), lambda t: (t, 0))]
    else:
        in_specs = [pl.BlockSpec((BE, 8), lambda t: (t, 0)), rtspec, wspec]
    in_specs += [pl.BlockSpec((32, H), lambda t: (0, 0)),
                 bspec, wspec, bspec, wspec, bspec, wspec, bspec]
    return pl.pallas_call(
        lambda *refs: _edge_body(has_v, refs),
        grid=(nb,),
        in_specs=in_specs,
        out_specs=[big, big, big, big],
        out_shape=[jax.ShapeDtypeStruct((E_PAD, H), jnp.float32)] * 4,
        interpret=interpret,
    )


def _node_body(first, *refs):
    if first:
        (zf, embp, ms, a0, a1, a2,
         us1t, usb1, us2t, usb2, uv1t, uvb1, uv2t, uvb2,
         out, outb) = refs
    else:
        (sv, ms, a0, a1, a2,
         us1t, usb1, us2t, usb2, uv1t, uvb1, uv2t, uvb2,
         out, outb) = refs

    def mlp(x, w1t, bb1, w2t, bb2):
        h = _silu(jnp.dot(x, w1t[...], preferred_element_type=jnp.float32)
                  + bb1[...])
        return jnp.dot(h, w2t[...], preferred_element_type=jnp.float32) + bb2[...]

    if first:
        zrow = zf[0]
        gid = lax.broadcasted_iota(jnp.int32, (H, BN), 0).astype(jnp.float32)
        onehot_t = (gid == zrow).astype(jnp.float32)
        s = lax.dot_general(onehot_t, embp[...], (((0,), (0,)), ((), ())),
                            preferred_element_type=jnp.float32)
    else:
        svv = sv[...]
        s = svv[:, 0:H]
    s_n = s + mlp(ms[0] + ms[1], us1t, usb1, us2t, usb2)
    outs = [s_n]
    for d, a in enumerate((a0, a1, a2)):
        upd = mlp(a[0] + a[1], uv1t, uvb1, uv2t, uvb2)
        if first:
            outs.append(upd)
        else:
            outs.append(svv[:, (d + 1) * H:(d + 2) * H] + upd)
    fused = jnp.concatenate(outs, axis=1)
    out[...] = fused
    fb = fused.astype(jnp.bfloat16)
    hi16 = lax.bitcast_convert_type(fb[:, :2 * H],
                                    jnp.uint16).astype(jnp.uint32)
    lo16 = lax.bitcast_convert_type(fb[:, 2 * H:],
                                    jnp.uint16).astype(jnp.uint32)
    outb[...] = lax.bitcast_convert_type((hi16 << 16) | lo16, jnp.int32)


def _tc_node(first, interpret=False):
    nb = N_PAD // BN
    big = pl.BlockSpec((BN, H), lambda t: (t, 0))
    fat = pl.BlockSpec((BN, 4 * H), lambda t: (t, 0))
    wspec = pl.BlockSpec((H, H), lambda t: (0, 0))
    bspec = pl.BlockSpec((1, H), lambda t: (0, 0))
    part = pl.BlockSpec((2, BN, H), lambda t: (0, t, 0))
    if first:
        lead = [pl.BlockSpec((1, 1, BN), lambda t: (t, 0, 0)), wspec]
    else:
        lead = [fat]
    return pl.pallas_call(
        functools.partial(_node_body, first),
        grid=(nb,),
        in_specs=lead + [part] * 4 + [wspec, bspec, wspec, bspec] * 2,
        out_specs=[fat, pl.BlockSpec((BN, 2 * H), lambda t: (t, 0))],
        out_shape=[jax.ShapeDtypeStruct((N_PAD, 4 * H), jnp.float32),
                   jax.ShapeDtypeStruct((N_PAD, 2 * H), jnp.int32)],
        interpret=interpret,
    )


def _readout_body(s, batch, wrot, brow, out):
    @pl.when(pl.program_id(0) == 0)
    def _():
        out[...] = jnp.zeros_like(out)
    per_atom = (jnp.dot(s[...], wrot[...], preferred_element_type=jnp.float32)
                + brow[...])
    bvec = batch[0]
    gid = lax.broadcasted_iota(jnp.int32, (NG, BN), 0)
    onehot = (gid == bvec).astype(jnp.float32)
    out[...] += jnp.dot(onehot, per_atom, preferred_element_type=jnp.float32)


def _tc_readout(interpret=False):
    nb = N_PAD // BN
    return pl.pallas_call(
        _readout_body,
        grid=(nb,),
        in_specs=[
            pl.BlockSpec((BN, H), lambda t: (t, 0)),
            pl.BlockSpec((1, 1, BN), lambda t: (t, 0, 0)),
            pl.BlockSpec((H, H), lambda t: (0, 0)),
            pl.BlockSpec((1, H), lambda t: (0, 0)),
        ],
        out_specs=pl.BlockSpec((NG, H), lambda t: (0, 0)),
        out_shape=jax.ShapeDtypeStruct((NG, H), jnp.float32),
        interpret=interpret,
    )


def _row(b):
    return b.reshape(1, H)


def kernel(z, pos, edge_index, batch, emb, layers, W_ro, b_ro):
    ei = edge_index[0].astype(jnp.int32)
    ej = edge_index[1].astype(jnp.int32)

    ei2d = jnp.concatenate(
        [ei, jnp.full((E_PAD - N_E,), N_N, jnp.int32)]).reshape(NROWS, CH)
    ej_pad = jnp.concatenate([ej, jnp.zeros((E_PAD - N_E,), jnp.int32)])
    ej2d = ej_pad.reshape(NROWS, CH)
    ej2d_g = ej_pad.reshape(GROWS, CH_G)

    geo = jnp.zeros((N_PAD, GEO_W), jnp.float32)
    geo = geo.at[:N_N, 0:3].set(pos).at[:N_N, 3].set(z.astype(jnp.float32))
    gi, gj = _sc_gather_geo()(geo, ei2d, ej2d)
    ef, r_t = _tc_geom()(gi, gj)

    z3 = jnp.concatenate(
        [z.astype(jnp.float32), jnp.zeros((N_PAD - N_N,), jnp.float32)]
    ).reshape(N_PAD // BN, 1, BN)
    emb_pad = jnp.zeros((H, H), jnp.float32).at[:emb.shape[0]].set(emb)

    gather = _sc_gather_fat()
    scatter4 = _sc_scatter4()
    edge1 = _tc_edge(False)
    edge2 = _tc_edge(True)

    sv = None
    for li, p in enumerate(layers):
        w1t = jnp.zeros((32, H), jnp.float32).at[:NRBF, :].set(p['phi'][0].T)
        wargs = (w1t, _row(p['phi'][1]), p['phi'][2].T, _row(p['phi'][3]),
                 p['Ws_W'].T, _row(p['Ws_b']), p['Wv_W'].T, _row(p['Wv_b']))
        if li == 0:
            ms_e, a0_e, a1_e, a2_e = edge1(ef, r_t, emb_pad, *wargs)
        else:
            svj = gather(svb, ej2d_g)
            ms_e, a0_e, a1_e, a2_e = edge2(ef, r_t, svj, *wargs)
        MS, A0, A1, A2 = scatter4(ms_e, a0_e, a1_e, a2_e, ei2d)
        lead = (z3, emb_pad) if li == 0 else (sv,)
        sv, svb = _tc_node(li == 0)(
            *lead, MS, A0, A1, A2,
            p['Us'][0].T, _row(p['Us'][1]), p['Us'][2].T, _row(p['Us'][3]),
            p['Uv'][0].T, _row(p['Uv'][1]), p['Uv'][2].T, _row(p['Uv'][3]))

    wrot = jnp.zeros((H, H), jnp.float32).at[:, :3].set(W_ro.T)
    brow = jnp.zeros((1, H), jnp.float32).at[0, :3].set(b_ro)
    batch3 = jnp.concatenate(
        [batch.astype(jnp.int32), jnp.full((N_PAD - N_N,), NG, jnp.int32)]
    ).reshape(N_PAD // BN, 1, BN)
    pred_pad = _tc_readout()(sv, batch3, wrot, brow)
    return pred_pad[:, :3]


# speedup vs baseline: 1.3964x; 1.0011x over previous
import functools

import jax
import jax.numpy as jnp
from jax import lax
from jax.experimental import pallas as pl
from jax.experimental.pallas import tpu as pltpu
from jax.experimental.pallas import tpu_sc as plsc

H = 128
NRBF = 20
CUT = 5.0
EPS = 1e-8
NG = 64
N_E = 320000
N_N = 10000
E_PAD = 327680
N_PAD = 10240

CH = 128
NROWS = E_PAD // CH
GW_ROWS = NROWS // 32
SC_ROWS = NROWS // 16
NPT = N_PAD // 16
BE = 512
BN = 1024


CH_G = 64
GROWS = E_PAD // CH_G
GW_FAT = GROWS // 32


def _sc_gather_fat():
    mesh = plsc.VectorSubcoreMesh(core_axis_name="c", subcore_axis_name="s")

    @functools.partial(
        pl.kernel, mesh=mesh,
        out_type=jax.ShapeDtypeStruct((E_PAD, 2 * H), jnp.int32),
        scratch_types=[
            pltpu.VMEM((GW_FAT, CH_G), jnp.int32),
        ] + [pltpu.VMEM((CH_G, 2 * H), jnp.int32)] * 4
          + [pltpu.SemaphoreType.DMA] * 8,
    )
    def k(table, idx_hbm, out, idx_v, b0, b1, b2, b3, *sems):
        bufs = (b0, b1, b2, b3)
        gsems = sems[:4]
        wsems = sems[4:]
        wid = lax.axis_index("c") * 16 + lax.axis_index("s")
        row0 = wid * GW_FAT
        pltpu.sync_copy(idx_hbm.at[pl.ds(row0, GW_FAT)], idx_v)

        def body(i0, _):
            for b in range(4):
                @pl.when(i0 > 0)
                def _(b=b):
                    pltpu.make_async_copy(
                        bufs[b], out.at[pl.ds((row0 + i0 + b) * CH_G, CH_G)],
                        wsems[b]).wait()
            hs = [pltpu.async_copy(table.at[idx_v.at[i0 + b]],
                                   bufs[b], gsems[b]) for b in range(4)]
            for b in range(4):
                hs[b].wait()
                pltpu.async_copy(
                    bufs[b], out.at[pl.ds((row0 + i0 + b) * CH_G, CH_G)],
                    wsems[b])
            return ()
        lax.fori_loop(0, GW_FAT // 4, lambda i, c: body(i * 4, c), ())
        for b in range(4):
            pltpu.make_async_copy(
                bufs[b], out.at[pl.ds(row0 * CH_G, CH_G)], wsems[b]).wait()

    return k


GEO_W = 128


def _sc_gather_geo():
    mesh = plsc.VectorSubcoreMesh(core_axis_name="c", subcore_axis_name="s")

    @functools.partial(
        pl.kernel, mesh=mesh,
        out_type=[jax.ShapeDtypeStruct((E_PAD, GEO_W), jnp.float32)] * 2,
        scratch_types=[
            pltpu.VMEM((GW_ROWS, CH), jnp.int32),
        ] + [pltpu.VMEM((CH, GEO_W), jnp.float32)] * 4
          + [pltpu.SemaphoreType.DMA] * 8,
    )
    def k(geo, ei_hbm, ej_hbm, oi, oj, idx_v, b0, b1, b2, b3, *sems):
        bufs = (b0, b1, b2, b3)
        gsems = sems[:4]
        wsems = sems[4:]
        wid = lax.axis_index("c") * 16 + lax.axis_index("s")
        row0 = wid * GW_ROWS
        for idxh, ob in ((ei_hbm, oi), (ej_hbm, oj)):
            pltpu.sync_copy(idxh.at[pl.ds(row0, GW_ROWS)], idx_v)

            def body(i0, _, ob=ob):
                for b in range(4):
                    @pl.when(i0 > 0)
                    def _(b=b, ob=ob):
                        pltpu.make_async_copy(
                            bufs[b], ob.at[pl.ds((row0 + i0 + b) * CH, CH)],
                            wsems[b]).wait()
                hs = [pltpu.async_copy(geo.at[idx_v.at[i0 + b]],
                                       bufs[b], gsems[b]) for b in range(4)]
                for b in range(4):
                    hs[b].wait()
                    pltpu.async_copy(
                        bufs[b], ob.at[pl.ds((row0 + i0 + b) * CH, CH)],
                        wsems[b])
                return ()
            lax.fori_loop(0, GW_ROWS // 4, lambda i, c: body(i * 4, c), ())
            for b in range(4):
                pltpu.make_async_copy(
                    bufs[b], ob.at[pl.ds(row0 * CH, CH)], wsems[b]).wait()

    return k


SC_ROWS_T = NROWS // 32


def _sc_scatter4():
    mesh = plsc.VectorSubcoreMesh(core_axis_name="c", subcore_axis_name="s")

    @functools.partial(
        pl.kernel, mesh=mesh,
        out_type=[jax.ShapeDtypeStruct((2, N_PAD, H), jnp.float32)] * 4,
        scratch_types=[
            pltpu.VMEM((SC_ROWS_T, CH), jnp.int32),
            pltpu.VMEM((CH, H), jnp.float32),
            pltpu.VMEM((CH, H), jnp.float32),
            pltpu.VMEM((32, H), jnp.float32),
            pltpu.VMEM_SHARED((N_PAD, H), jnp.float32),
            pltpu.SemaphoreType.DMA,
            pltpu.SemaphoreType.DMA,
        ],
    )
    def k(v0, v1, v2, v3, idx_hbm, o0, o1, o2, o3,
          idx_v, rb0, rb1, zbuf, acc, sm0, sm1):
        bufs = (rb0, rb1)
        sems = (sm0, sm1)
        c = lax.axis_index("c")
        s = lax.axis_index("s")
        row0 = c * (NROWS // 2) + s * SC_ROWS_T
        pltpu.sync_copy(idx_hbm.at[pl.ds(row0, SC_ROWS_T)], idx_v)

        def zfill(t, _):
            i = t // 8
            j = (t % 8) * 16
            zbuf[i, pl.ds(j, 16)] = jnp.zeros((16,), jnp.float32)
            return ()
        lax.fori_loop(0, 32 * 8, zfill, ())

        nbase = s * NPT

        def do_array(vals, out):
            def zb(t, _):
                pltpu.sync_copy(zbuf, acc.at[pl.ds(nbase + t * 32, 32)])
                return ()
            lax.fori_loop(0, NPT // 32, zb, ())
            plsc.subcore_barrier()

            def body(t0, _, vals=vals):
                hs = [pltpu.async_copy(
                          vals.at[pl.ds((row0 + t0 + b) * CH, CH)],
                          bufs[b], sems[b])
                      for b in range(2)]
                for b in range(2):
                    hs[b].wait()
                    pltpu.sync_copy(bufs[b], acc.at[idx_v.at[t0 + b]],
                                    add=True)
                return ()
            lax.fori_loop(0, SC_ROWS_T // 2, lambda i, cc: body(i * 2, cc), ())
            plsc.subcore_barrier()
            pltpu.sync_copy(acc.at[pl.ds(nbase, NPT)],
                            out.at[c].at[pl.ds(nbase, NPT)])

        do_array(v0, o0)
        do_array(v1, o1)
        do_array(v2, o2)
        do_array(v3, o3)

    return k


def _silu(x):
    return x * (1.0 / (1.0 + jnp.exp(-x)))


def _geom_body(gi, gj, ef_o, rt_o):
    a = gi[...]
    b = gj[...]
    dx = b[:, 0:1] - a[:, 0:1]
    dy = b[:, 1:2] - a[:, 1:2]
    dz = b[:, 2:3] - a[:, 2:3]
    dist = jnp.sqrt(dx * dx + dy * dy + dz * dz)
    dsafe = jnp.maximum(dist, EPS)
    inv = 1.0 / dsafe
    ef_o[...] = jnp.concatenate(
        [dx * inv, dy * inv, dz * inv, dsafe, b[:, 3:4],
         jnp.zeros((BE, 3), jnp.float32)], axis=1)
    rt_o[...] = jnp.transpose(dsafe)


def _tc_geom(interpret=False):
    nb = E_PAD // BE
    gspec = pl.BlockSpec((BE, GEO_W), lambda t: (t, 0))
    return pl.pallas_call(
        _geom_body,
        grid=(nb,),
        in_specs=[gspec, gspec],
        out_specs=[pl.BlockSpec((BE, 8), lambda t: (t, 0)),
                   pl.BlockSpec((1, BE), lambda t: (0, t))],
        out_shape=[jax.ShapeDtypeStruct((E_PAD, 8), jnp.float32),
                   jax.ShapeDtypeStruct((1, E_PAD), jnp.float32)],
        interpret=interpret,
    )


def _edge_body(has_v, refs):
    if has_v:
        (ef, rt, pk, w1t, b1, w2t, b2, wst, bs, wvt, bv,
         ms_o, a0_o, a1_o, a2_o) = refs
    else:
        (ef, rt, embp, w1t, b1, w2t, b2, wst, bs, wvt, bv,
         ms_o, a0_o, a1_o, a2_o) = refs
    e = ef[...]
    u0 = e[:, 0:1]
    u1 = e[:, 1:2]
    u2 = e[:, 2:3]
    if not has_v:
        kcol = lax.broadcasted_iota(jnp.int32, (BE, H), 1).astype(jnp.float32)
        onehot = (e[:, 4:5] == kcol).astype(jnp.float32)
        sj_v = jnp.dot(onehot, embp[...], preferred_element_type=jnp.float32)
    rr = rt[...]
    x = rr * (jnp.pi / CUT)
    c1 = jnp.cos(x)
    s1 = jnp.sin(x)
    cv = jnp.where(rr < CUT, 0.5 * (c1 + 1.0), 0.0)
    scale = cv / rr
    two_c1 = 2.0 * c1
    p_prev = jnp.zeros((1, BE), jnp.float32)
    p_cur = s1 * scale
    rows = [p_cur]
    for _ in range(NRBF - 1):
        p_prev, p_cur = p_cur, two_c1 * p_cur - p_prev
        rows.append(p_cur)
    rows.append(jnp.zeros((12, BE), jnp.float32))
    rbf_t = jnp.concatenate(rows, axis=0)
    h1 = _silu(lax.dot_general(rbf_t, w1t[...],
                               (((0,), (0,)), ((), ())),
                               preferred_element_type=jnp.float32)
               + b1[...])
    w = jnp.dot(h1, w2t[...], preferred_element_type=jnp.float32) + b2[...]
    if has_v:
        ui = lax.bitcast_convert_type(pk[...], jnp.uint32)
        hi = lax.bitcast_convert_type((ui >> 16).astype(jnp.uint16),
                                      jnp.bfloat16).astype(jnp.float32)
        lo = lax.bitcast_convert_type((ui & 0xFFFF).astype(jnp.uint16),
                                      jnp.bfloat16).astype(jnp.float32)
        sj_v = hi[:, :H]
    ms = (jnp.dot(sj_v, wst[...], preferred_element_type=jnp.float32)
          + bs[...]) * w
    if has_v:
        proj = u0 * hi[:, H:] + u1 * lo[:, :H] + u2 * lo[:, H:]
        mv = (jnp.dot(proj, wvt[...], preferred_element_type=jnp.float32)
              + bv[...]) * w
    else:
        mv = bv[...] * w
    ms_o[...] = ms
    a0_o[...] = u0 * mv
    a1_o[...] = u1 * mv
    a2_o[...] = u2 * mv


def _tc_edge(has_v, interpret=False):
    nb = E_PAD // BE
    big = pl.BlockSpec((BE, H), lambda t: (t, 0))
    wspec = pl.BlockSpec((H, H), lambda t: (0, 0))
    bspec = pl.BlockSpec((1, H), lambda t: (0, 0))
    rtspec = pl.BlockSpec((1, BE), lambda t: (0, t))
    if has_v:
        in_specs = [pl.BlockSpec((BE, 8), lambda t: (t, 0)), rtspec,
                    pl.BlockSpec((BE, 2 * H), lambda t: (t, 0))]
    else:
        in_specs = [pl.BlockSpec((BE, 8), lambda t: (t, 0)), rtspec, wspec]
    in_specs += [pl.BlockSpec((32, H), lambda t: (0, 0)),
                 bspec, wspec, bspec, wspec, bspec, wspec, bspec]
    return pl.pallas_call(
        lambda *refs: _edge_body(has_v, refs),
        grid=(nb,),
        in_specs=in_specs,
        out_specs=[big, big, big, big],
        out_shape=[jax.ShapeDtypeStruct((E_PAD, H), jnp.float32)] * 4,
        interpret=interpret,
    )


def _node_body(first, *refs):
    if first:
        (zf, embp, ms, a0, a1, a2,
         us1t, usb1, us2t, usb2, uv1t, uvb1, uv2t, uvb2,
         out, outb) = refs
    else:
        (sv, ms, a0, a1, a2,
         us1t, usb1, us2t, usb2, uv1t, uvb1, uv2t, uvb2,
         out, outb) = refs

    def mlp(x, w1t, bb1, w2t, bb2):
        h = _silu(jnp.dot(x, w1t[...], preferred_element_type=jnp.float32)
                  + bb1[...])
        return jnp.dot(h, w2t[...], preferred_element_type=jnp.float32) + bb2[...]

    if first:
        zrow = zf[0]
        gid = lax.broadcasted_iota(jnp.int32, (H, BN), 0).astype(jnp.float32)
        onehot_t = (gid == zrow).astype(jnp.float32)
        s = lax.dot_general(onehot_t, embp[...], (((0,), (0,)), ((), ())),
                            preferred_element_type=jnp.float32)
    else:
        svv = sv[...]
        s = svv[:, 0:H]
    s_n = s + mlp(ms[0] + ms[1], us1t, usb1, us2t, usb2)
    outs = [s_n]
    for d, a in enumerate((a0, a1, a2)):
        upd = mlp(a[0] + a[1], uv1t, uvb1, uv2t, uvb2)
        if first:
            outs.append(upd)
        else:
            outs.append(svv[:, (d + 1) * H:(d + 2) * H] + upd)
    fused = jnp.concatenate(outs, axis=1)
    out[...] = fused
    fb = fused.astype(jnp.bfloat16)
    hi16 = lax.bitcast_convert_type(fb[:, :2 * H],
                                    jnp.uint16).astype(jnp.uint32)
    lo16 = lax.bitcast_convert_type(fb[:, 2 * H:],
                                    jnp.uint16).astype(jnp.uint32)
    outb[...] = lax.bitcast_convert_type((hi16 << 16) | lo16, jnp.int32)


def _tc_node(first, interpret=False):
    nb = N_PAD // BN
    big = pl.BlockSpec((BN, H), lambda t: (t, 0))
    fat = pl.BlockSpec((BN, 4 * H), lambda t: (t, 0))
    wspec = pl.BlockSpec((H, H), lambda t: (0, 0))
    bspec = pl.BlockSpec((1, H), lambda t: (0, 0))
    part = pl.BlockSpec((2, BN, H), lambda t: (0, t, 0))
    if first:
        lead = [pl.BlockSpec((1, 1, BN), lambda t: (t, 0, 0)), wspec]
    else:
        lead = [fat]
    return pl.pallas_call(
        functools.partial(_node_body, first),
        grid=(nb,),
        in_specs=lead + [part] * 4 + [wspec, bspec, wspec, bspec] * 2,
        out_specs=[fat, pl.BlockSpec((BN, 2 * H), lambda t: (t, 0))],
        out_shape=[jax.ShapeDtypeStruct((N_PAD, 4 * H), jnp.float32),
                   jax.ShapeDtypeStruct((N_PAD, 2 * H), jnp.int32)],
        interpret=interpret,
    )


def _readout_body(s, batch, wrot, brow, out):
    @pl.when(pl.program_id(0) == 0)
    def _():
        out[...] = jnp.zeros_like(out)
    per_atom = (jnp.dot(s[...], wrot[...], preferred_element_type=jnp.float32)
                + brow[...])
    bvec = batch[0]
    gid = lax.broadcasted_iota(jnp.int32, (NG, BN), 0)
    onehot = (gid == bvec).astype(jnp.float32)
    out[...] += jnp.dot(onehot, per_atom, preferred_element_type=jnp.float32)


def _tc_readout(interpret=False):
    nb = N_PAD // BN
    return pl.pallas_call(
        _readout_body,
        grid=(nb,),
        in_specs=[
            pl.BlockSpec((BN, H), lambda t: (t, 0)),
            pl.BlockSpec((1, 1, BN), lambda t: (t, 0, 0)),
            pl.BlockSpec((H, H), lambda t: (0, 0)),
            pl.BlockSpec((1, H), lambda t: (0, 0)),
        ],
        out_specs=pl.BlockSpec((NG, H), lambda t: (0, 0)),
        out_shape=jax.ShapeDtypeStruct((NG, H), jnp.float32),
        interpret=interpret,
    )


def _row(b):
    return b.reshape(1, H)


def kernel(z, pos, edge_index, batch, emb, layers, W_ro, b_ro):
    ei = edge_index[0].astype(jnp.int32)
    ej = edge_index[1].astype(jnp.int32)

    ei2d = jnp.concatenate(
        [ei, jnp.full((E_PAD - N_E,), N_N, jnp.int32)]).reshape(NROWS, CH)
    ej_pad = jnp.concatenate([ej, jnp.zeros((E_PAD - N_E,), jnp.int32)])
    ej2d = ej_pad.reshape(NROWS, CH)
    ej2d_g = ej_pad.reshape(GROWS, CH_G)

    geo = jnp.zeros((N_PAD, GEO_W), jnp.float32)
    geo = geo.at[:N_N, 0:3].set(pos).at[:N_N, 3].set(z.astype(jnp.float32))
    gi, gj = _sc_gather_geo()(geo, ei2d, ej2d)
    ef, r_t = _tc_geom()(gi, gj)

    z3 = jnp.concatenate(
        [z.astype(jnp.float32), jnp.zeros((N_PAD - N_N,), jnp.float32)]
    ).reshape(N_PAD // BN, 1, BN)
    emb_pad = jnp.zeros((H, H), jnp.float32).at[:emb.shape[0]].set(emb)

    gather = _sc_gather_fat()
    scatter4 = _sc_scatter4()
    edge1 = _tc_edge(False)
    edge2 = _tc_edge(True)

    sv = None
    for li, p in enumerate(layers):
        w1t = jnp.zeros((32, H), jnp.float32).at[:NRBF, :].set(p['phi'][0].T)
        wargs = (w1t, _row(p['phi'][1]), p['phi'][2].T, _row(p['phi'][3]),
                 p['Ws_W'].T, _row(p['Ws_b']), p['Wv_W'].T, _row(p['Wv_b']))
        if li == 0:
            ms_e, a0_e, a1_e, a2_e = edge1(ef, r_t, emb_pad, *wargs)
        else:
            svj = gather(svb, ej2d_g)
            ms_e, a0_e, a1_e, a2_e = edge2(ef, r_t, svj, *wargs)
        MS, A0, A1, A2 = scatter4(ms_e, a0_e, a1_e, a2_e, ei2d)
        lead = (z3, emb_pad) if li == 0 else (sv,)
        sv, svb = _tc_node(li == 0)(
            *lead, MS, A0, A1, A2,
            p['Us'][0].T, _row(p['Us'][1]), p['Us'][2].T, _row(p['Us'][3]),
            p['Uv'][0].T, _row(p['Uv'][1]), p['Uv'][2].T, _row(p['Uv'][3]))

    wrot = jnp.zeros((H, H), jnp.float32).at[:, :3].set(W_ro.T)
    brow = jnp.zeros((1, H), jnp.float32).at[0, :3].set(b_ro)
    batch3 = jnp.concatenate(
        [batch.astype(jnp.int32), jnp.full((N_PAD - N_N,), NG, jnp.int32)]
    ).reshape(N_PAD // BN, 1, BN)
    pred_pad = _tc_readout()(sv, batch3, wrot, brow)
    return pred_pad[:, :3]
